# Initial kernel scaffold; baseline (speedup 1.0000x reference)
import functools

import jax
import jax.numpy as jnp
from jax import lax
from jax.experimental import pallas as pl
from jax.experimental.pallas import tpu as pltpu
from jax.experimental.pallas import tpu_sc as plsc

N_BINS = 2048
HIDDEN = 64
BATCH = 16384
FIELDS = 100
TOTAL = BATCH * FIELDS

NC = 2
NS = 16
NW = NC * NS
PER_W = TOTAL // NW
CHUNK = 512
NCHUNK = PER_W // CHUNK
SUB = 128
G = CHUNK // SUB
LANES = 16

_mesh = plsc.VectorSubcoreMesh(core_axis_name="c", subcore_axis_name="s")


@functools.partial(
    pl.kernel,
    mesh=_mesh,
    out_type=jax.ShapeDtypeStruct((TOTAL, HIDDEN), jnp.float32),
    scratch_types=[
        pltpu.VMEM((N_BINS,), jnp.float32),
        pltpu.VMEM((CHUNK,), jnp.float32),
        pltpu.VMEM((G, SUB), jnp.int32),
        pltpu.VMEM((CHUNK, HIDDEN), jnp.float32),
        pltpu.SemaphoreType.DMA,
    ],
)
def _emb_kernel(x_hbm, bounds_hbm, table_hbm, out_hbm,
                bounds_v, x_v, idx_v, rows_v, sem):
    wid = lax.axis_index("s") * NC + lax.axis_index("c")
    base = wid * PER_W
    pltpu.sync_copy(bounds_hbm, bounds_v)

    def chunk_body(ci, carry):
        off = base + ci * CHUNK
        pltpu.sync_copy(x_hbm.at[pl.ds(off, CHUNK)], x_v)
        for g in range(G):
            for v in range(SUB // LANES):
                o = g * SUB + v * LANES
                xv = x_v[pl.ds(o, LANES)]
                k = jnp.clip((xv * jnp.float32(2046.0)).astype(jnp.int32),
                             1, N_BINS - 3)
                b0 = plsc.load_gather(bounds_v, [k - 1])
                b1 = plsc.load_gather(bounds_v, [k])
                b2 = plsc.load_gather(bounds_v, [k + 1])
                one = jnp.int32(1)
                zero = jnp.int32(0)
                idx = ((k - 1)
                       + jnp.where(b0 < xv, one, zero)
                       + jnp.where(b1 < xv, one, zero)
                       + jnp.where(b2 < xv, one, zero))
                idx_v[g, pl.ds(v * LANES, LANES)] = idx
        copies = [
            pltpu.async_copy(table_hbm.at[idx_v.at[g]],
                             rows_v.at[pl.ds(g * SUB, SUB)], sem)
            for g in range(G)
        ]
        for c in copies:
            c.wait()
        pltpu.sync_copy(rows_v, out_hbm.at[pl.ds(off, CHUNK)])
        return carry

    lax.fori_loop(0, NCHUNK, chunk_body, 0)


def kernel(x, table):
    bounds = jnp.linspace(0.0, 1.0, N_BINS - 1, dtype=jnp.float32)
    bounds = jnp.concatenate([bounds, jnp.ones((1,), jnp.float32)])
    out = _emb_kernel(x.reshape(TOTAL), bounds, table)
    return out.reshape(BATCH, FIELDS, HIDDEN)


# speedup vs baseline: 125.4009x; 125.4009x over previous
import functools

import jax
import jax.numpy as jnp
from jax import lax
from jax.experimental import pallas as pl
from jax.experimental.pallas import tpu as pltpu
from jax.experimental.pallas import tpu_sc as plsc

N_BINS = 2048
HIDDEN = 64
BATCH = 16384
FIELDS = 100
TOTAL = BATCH * FIELDS

NC = 2
NS = 16
NW = NC * NS
PER_W = TOTAL // NW
CHUNK = 512
NCHUNK = PER_W // CHUNK
SUB = 128
G = CHUNK // SUB
LANES = 16

_mesh = plsc.VectorSubcoreMesh(core_axis_name="c", subcore_axis_name="s")


@functools.partial(
    pl.kernel,
    mesh=_mesh,
    out_type=jax.ShapeDtypeStruct((TOTAL, HIDDEN), jnp.float32),
    scratch_types=[
        pltpu.VMEM((N_BINS,), jnp.float32),
        pltpu.VMEM((CHUNK,), jnp.float32),
        pltpu.VMEM((G, SUB), jnp.int32),
        pltpu.VMEM((CHUNK, HIDDEN), jnp.float32),
        pltpu.SemaphoreType.DMA,
    ],
    compiler_params=pltpu.CompilerParams(
        needs_layout_passes=False, use_tc_tiling_on_sc=False),
)
def _emb_kernel(x_hbm, bounds_hbm, table_hbm, out_hbm,
                bounds_v, x_v, idx_v, rows_v, sem):
    wid = lax.axis_index("s") * NC + lax.axis_index("c")
    base = wid * PER_W
    pltpu.sync_copy(bounds_hbm, bounds_v)

    def chunk_body(ci, carry):
        off = base + ci * CHUNK
        pltpu.sync_copy(x_hbm.at[pl.ds(off, CHUNK)], x_v)
        for g in range(G):
            for v in range(SUB // LANES):
                o = g * SUB + v * LANES
                xv = x_v[pl.ds(o, LANES)]
                k = jnp.clip((xv * jnp.float32(2046.0)).astype(jnp.int32),
                             1, N_BINS - 3)
                b0 = plsc.load_gather(bounds_v, [k - 1])
                b1 = plsc.load_gather(bounds_v, [k])
                b2 = plsc.load_gather(bounds_v, [k + 1])
                one = jnp.int32(1)
                zero = jnp.int32(0)
                idx = ((k - 1)
                       + jnp.where(b0 < xv, one, zero)
                       + jnp.where(b1 < xv, one, zero)
                       + jnp.where(b2 < xv, one, zero))
                idx_v[g, pl.ds(v * LANES, LANES)] = idx
        copies = [
            pltpu.async_copy(table_hbm.at[idx_v.at[g]],
                             rows_v.at[pl.ds(g * SUB, SUB)], sem)
            for g in range(G)
        ]
        for c in copies:
            c.wait()
        pltpu.sync_copy(rows_v, out_hbm.at[pl.ds(off, CHUNK)])
        return carry

    lax.fori_loop(0, NCHUNK, chunk_body, 0)


def kernel(x, table):
    bounds = jnp.linspace(0.0, 1.0, N_BINS - 1, dtype=jnp.float32)
    bounds = jnp.concatenate([bounds, jnp.ones((1,), jnp.float32)])
    out = _emb_kernel(x.reshape(TOTAL), bounds, table)
    return out.reshape(BATCH, FIELDS, HIDDEN)
